# Initial kernel scaffold; baseline (speedup 1.0000x reference)
import jax
import jax.numpy as jnp
from jax.experimental import pallas as pl


def kernel(query, reference_points, value, spatial_shapes, level_start_index, W_value, b_value, W_off, b_off, W_attn, b_attn, W_out, b_out):
    raise NotImplementedError("write your pallas kernel here")


# speedup vs baseline: 37.2709x; 37.2709x over previous
import functools

import numpy as np
import jax
import jax.numpy as jnp
from jax import lax
from jax.experimental import pallas as pl
from jax.experimental.pallas import tpu as pltpu
from jax.experimental.pallas import tpu_sc as plsc

_SPATIAL = [(64, 64), (32, 32), (16, 16), (8, 8)]
_B, _Q, _D = 8, 900, 256
_H, _HD = 8, 32
_L, _P = 4, 4
_S = sum(h * w for h, w in _SPATIAL)
_NQ = _B * _Q
_N = _NQ * _H
_QBLK = 400
_NBLK = _NQ // _QBLK
_RL = [(h + 2) * (w + 2) for h, w in _SPATIAL]
_NW = 32
_ROWS_PER_W = _NQ // _NW

_COL_L = np.repeat(np.arange(_L), 32)
_COL_HEAD = np.tile(np.repeat(np.arange(_H), _P), _L)
_WVEC = np.array([_SPATIAL[l][1] for l in _COL_L], np.float32)
_HVEC = np.array([_SPATIAL[l][0] for l in _COL_L], np.float32)
_PWVEC = _WVEC + 2.0
_RVEC = np.array([_RL[l] for l in _COL_L], np.float32)
_HEADVEC = _COL_HEAD.astype(np.float32)

_ELX = np.zeros((2 * _L, 128), np.float32)
_ELY = np.zeros((2 * _L, 128), np.float32)
for _c in range(128):
    _ELX[2 * _COL_L[_c] + 0, _c] = 1.0
    _ELY[2 * _COL_L[_c] + 1, _c] = 1.0

_G = np.zeros((128, 128), np.float32)
for _i in range(128):
    for _j in range(128):
        if _COL_HEAD[_i] == _COL_HEAD[_j]:
            _G[_i, _j] = 1.0

_SINT = np.zeros((128, 128), np.float32)
for _c in range(4):
    for _hp in range(32):
        _SINT[_c * 32 + _hp, _hp * 4 + _c] = 1.0


def _tables_kernel(val_ref, wvt_ref, bv_ref, *out_refs):
    v = jnp.dot(val_ref[0], wvt_ref[:], preferred_element_type=jnp.float32, precision=lax.Precision.HIGHEST)
    v = v + bv_ref[:]
    start = 0
    for lvl, (hh, ww) in enumerate(_SPATIAL):
        vl = v[start:start + hh * ww].reshape(hh, ww, _D)
        start += hh * ww
        zc = jnp.zeros((hh, 1, _D), jnp.float32)
        vl = jnp.concatenate([zc, vl, zc], axis=1)
        zr = jnp.zeros((1, ww + 2, _D), jnp.float32)
        vl = jnp.concatenate([zr, vl, zr], axis=0)
        out_refs[lvl][0] = vl.reshape((hh + 2) * (ww + 2), _D)


def _build_tables(value, W_value, b_value):
    wvt = W_value.T
    bv = b_value.reshape(1, _D)
    out_shapes = [jax.ShapeDtypeStruct((_B, r, _D), jnp.float32)
                  for r in _RL]
    out_specs = [pl.BlockSpec((1, r, _D), lambda b: (b, 0, 0))
                 for r in _RL]
    tables = pl.pallas_call(
        _tables_kernel,
        grid=(_B,),
        in_specs=[
            pl.BlockSpec((1, _S, _D), lambda b: (b, 0, 0)),
            pl.BlockSpec((_D, _D), lambda b: (0, 0)),
            pl.BlockSpec((1, _D), lambda b: (0, 0)),
        ],
        out_specs=out_specs,
        out_shape=out_shapes,
    )(value, wvt, bv)
    return [t.reshape(_B * r * _H, _HD) for t, r in zip(tables, _RL)]


def _index_kernel(q_ref, ref_ref, woff_ref, boff_ref, wattn_ref, battn_ref,
                  g_ref, elx_ref, ely_ref, sint_ref, cvec_ref,
                  idx_ref, wts_ref):
    i = pl.program_id(0)
    qb = q_ref[:]
    off = jnp.dot(qb, woff_ref[:], preferred_element_type=jnp.float32, precision=lax.Precision.HIGHEST)
    off = off + boff_ref[:]
    ox = off[:, :128]
    oy = off[:, 128:]
    logit = jnp.dot(qb, wattn_ref[:], preferred_element_type=jnp.float32, precision=lax.Precision.HIGHEST)
    logit = logit + battn_ref[:]
    e = jnp.exp(logit)
    ssum = jnp.dot(e, g_ref[:], preferred_element_type=jnp.float32, precision=lax.Precision.HIGHEST)
    aw = e / ssum

    rx = jnp.dot(ref_ref[:], elx_ref[:],
                 preferred_element_type=jnp.float32, precision=lax.Precision.HIGHEST)
    ry = jnp.dot(ref_ref[:], ely_ref[:],
                 preferred_element_type=jnp.float32, precision=lax.Precision.HIGHEST)

    cvec = cvec_ref[:]
    wv = cvec[0:1, :]
    hv = cvec[1:2, :]
    gx = rx * wv + ox - 0.5
    gy = ry * hv + oy - 0.5
    x0 = jnp.floor(gx)
    y0 = jnp.floor(gy)
    wx1 = gx - x0
    wx0 = 1.0 - wx1
    wy1 = gy - y0
    wy0 = 1.0 - wy1
    px = jnp.clip(x0, -1.0, wv - 1.0)
    py = jnp.clip(y0, -1.0, hv - 1.0)
    vx = (x0 == px).astype(jnp.float32)
    vy = (y0 == py).astype(jnp.float32)
    wx0 = wx0 * vx
    wx1 = wx1 * vx
    wy0 = wy0 * vy
    wy1 = wy1 * vy

    wts_ref[:] = jnp.stack(
        [aw * wy0 * wx0, aw * wy0 * wx1, aw * wy1 * wx0, aw * wy1 * wx1],
        axis=1)

    rowf = (jnp.float32(i * _QBLK)
            + lax.broadcasted_iota(jnp.int32, (_QBLK, 128), 0
                                   ).astype(jnp.float32))
    bidx = jnp.floor(rowf / jnp.float32(_Q))
    pwv = cvec[2:3, :]
    base = (bidx * 8.0 * cvec[3:4, :]
            + ((py + 1.0) * pwv + (px + 1.0)) * 8.0 + cvec[4:5, :])
    sint = sint_ref[:]
    levels = []
    for lvl in range(_L):
        sl = slice(lvl * 32, (lvl + 1) * 32)
        pw8 = pwv[:, sl] * 8.0
        cat = jnp.concatenate(
            [base[:, sl], base[:, sl] + 8.0,
             base[:, sl] + pw8, base[:, sl] + pw8 + 8.0],
            axis=1)
        levels.append(jnp.dot(cat, sint,
                              preferred_element_type=jnp.float32, precision=lax.Precision.HIGHEST))
    idx_ref[:] = jnp.stack(levels, axis=1).astype(jnp.int32)


def _build_index(query, reference_points, W_off, b_off, W_attn, b_attn):
    perm_off = np.zeros(2 * _L * _H * _P, np.int64)
    for hd in range(_H):
        for lvl in range(_L):
            for p in range(_P):
                for xy in range(2):
                    src = ((hd * _L + lvl) * _P + p) * 2 + xy
                    dst = xy * 128 + lvl * 32 + hd * 4 + p
                    perm_off[dst] = src
    perm_attn = np.zeros(_L * _H * _P, np.int64)
    for hd in range(_H):
        for lvl in range(_L):
            for p in range(_P):
                src = (hd * _L + lvl) * _P + p
                dst = lvl * 32 + hd * 4 + p
                perm_attn[dst] = src
    woff_t = W_off[perm_off].T
    boff = b_off[perm_off].reshape(1, 256)
    wattn_t = W_attn[perm_attn].T
    battn = b_attn[perm_attn].reshape(1, 128)
    qf = query.reshape(_NQ, _D)
    rf = reference_points.reshape(_NQ, 2 * _L)
    cvec = np.zeros((8, 128), np.float32)
    cvec[0], cvec[1], cvec[2] = _WVEC, _HVEC, _PWVEC
    cvec[3], cvec[4] = _RVEC, _HEADVEC
    return pl.pallas_call(
        _index_kernel,
        grid=(_NBLK,),
        in_specs=[
            pl.BlockSpec((_QBLK, _D), lambda i: (i, 0)),
            pl.BlockSpec((_QBLK, 2 * _L), lambda i: (i, 0)),
            pl.BlockSpec((_D, _D), lambda i: (0, 0)),
            pl.BlockSpec((1, _D), lambda i: (0, 0)),
            pl.BlockSpec((_D, 128), lambda i: (0, 0)),
            pl.BlockSpec((1, 128), lambda i: (0, 0)),
            pl.BlockSpec((128, 128), lambda i: (0, 0)),
            pl.BlockSpec((2 * _L, 128), lambda i: (0, 0)),
            pl.BlockSpec((2 * _L, 128), lambda i: (0, 0)),
            pl.BlockSpec((128, 128), lambda i: (0, 0)),
            pl.BlockSpec((8, 128), lambda i: (0, 0)),
        ],
        out_specs=[
            pl.BlockSpec((_QBLK, _L, 128), lambda i: (i, 0, 0)),
            pl.BlockSpec((_QBLK, 4, 128), lambda i: (i, 0, 0)),
        ],
        out_shape=[
            jax.ShapeDtypeStruct((_NQ, _L, 128), jnp.int32),
            jax.ShapeDtypeStruct((_NQ, 4, 128), jnp.float32),
        ],
    )(qf, rf, woff_t, boff, wattn_t, battn,
      _G, _ELX, _ELY, _SINT, cvec)


def _sc_kernel(t0, t1, t2, t3, idx_hbm, wts_hbm, out_hbm,
               idx_v, w_v, dst_v, out_v, sem):
    tabs = (t0, t1, t2, t3)
    wid = lax.axis_index("s") * 2 + lax.axis_index("c")
    qbase = wid * _ROWS_PER_W

    def chunk(i, _):
        qrow = qbase + i
        pltpu.sync_copy(idx_hbm.at[qrow], idx_v)
        pltpu.sync_copy(wts_hbm.at[qrow], w_v)
        cps = [pltpu.async_copy(tabs[l].at[idx_v.at[l]], dst_v.at[l], sem)
               for l in range(_L)]
        for cp in cps:
            cp.wait()

        wvecs = {}
        for lvl in range(_L):
            for c in range(4):
                for hg in range(2):
                    wvecs[(lvl, c, hg)] = w_v[c, pl.ds(lvl * 32 + hg * 16, 16)]
        for hd in range(_H):
            hg, j = hd // 4, hd % 4
            acc0 = jnp.zeros((16,), jnp.float32)
            acc1 = jnp.zeros((16,), jnp.float32)
            for lvl in range(_L):
                for p in range(_P):
                    for c in range(4):
                        r = hd * 16 + p * 4 + c
                        w = wvecs[(lvl, c, hg)][j * 4 + p]
                        wb = jnp.full((16,), w, jnp.float32)
                        acc0 = acc0 + wb * dst_v[lvl, r, pl.ds(0, 16)]
                        acc1 = acc1 + wb * dst_v[lvl, r, pl.ds(16, 16)]
            out_v[hd, pl.ds(0, 16)] = acc0
            out_v[hd, pl.ds(16, 16)] = acc1
        pltpu.sync_copy(out_v, out_hbm.at[pl.ds(qrow * _H, _H)])
        return 0

    lax.fori_loop(0, _ROWS_PER_W, chunk, 0)


def _sc_gather(tables, idx, wts):
    mesh = plsc.VectorSubcoreMesh(core_axis_name="c", subcore_axis_name="s")
    run = pl.kernel(
        _sc_kernel,
        out_type=jax.ShapeDtypeStruct((_N, _HD), jnp.float32),
        mesh=mesh,
        scratch_types=[
            pltpu.VMEM((_L, 128), jnp.int32),
            pltpu.VMEM((4, 128), jnp.float32),
            pltpu.VMEM((_L, 128, _HD), jnp.float32),
            pltpu.VMEM((_H, _HD), jnp.float32),
            pltpu.SemaphoreType.DMA,
        ],
        compiler_params=pltpu.CompilerParams(use_tc_tiling_on_sc=False),
    )
    return run(*tables, idx, wts)


def _proj_kernel(x_ref, w_ref, b_ref, o_ref):
    o_ref[:] = (jnp.dot(x_ref[:], w_ref[:], preferred_element_type=jnp.float32, precision=lax.Precision.HIGHEST)
                + b_ref[:])


def _out_proj(sampled, W_out, b_out):
    x = sampled.reshape(_NQ, _D)
    out = pl.pallas_call(
        _proj_kernel,
        grid=(_NBLK,),
        in_specs=[
            pl.BlockSpec((_QBLK, _D), lambda i: (i, 0)),
            pl.BlockSpec((_D, _D), lambda i: (0, 0)),
            pl.BlockSpec((1, _D), lambda i: (0, 0)),
        ],
        out_specs=pl.BlockSpec((_QBLK, _D), lambda i: (i, 0)),
        out_shape=jax.ShapeDtypeStruct((_NQ, _D), jnp.float32),
    )(x, W_out.T, b_out.reshape(1, _D))
    return out.reshape(_B, _Q, _D)


def kernel(query, reference_points, value, spatial_shapes, level_start_index,
           W_value, b_value, W_off, b_off, W_attn, b_attn, W_out, b_out):
    del spatial_shapes, level_start_index
    tables = _build_tables(value, W_value, b_value)
    idx, wts = _build_index(query, reference_points, W_off, b_off,
                            W_attn, b_attn)
    sampled = _sc_gather(tables, idx, wts)
    return _out_proj(sampled, W_out, b_out)


# speedup vs baseline: 90.1661x; 2.4192x over previous
import functools

import numpy as np
import jax
import jax.numpy as jnp
from jax import lax
from jax.experimental import pallas as pl
from jax.experimental.pallas import tpu as pltpu
from jax.experimental.pallas import tpu_sc as plsc

_SPATIAL = [(64, 64), (32, 32), (16, 16), (8, 8)]
_B, _Q, _D = 8, 900, 256
_H, _HD = 8, 32
_L, _P = 4, 4
_S = sum(h * w for h, w in _SPATIAL)
_NQ = _B * _Q
_N = _NQ * _H
_QBLK = 400
_NBLK = _NQ // _QBLK
_RL = [(h + 2) * (w + 2) for h, w in _SPATIAL]
_NW = 32
_ROWS_PER_W = _NQ // _NW

_COL_L = np.repeat(np.arange(_L), 32)
_COL_HEAD = np.tile(np.repeat(np.arange(_H), _P), _L)
_WVEC = np.array([_SPATIAL[l][1] for l in _COL_L], np.float32)
_HVEC = np.array([_SPATIAL[l][0] for l in _COL_L], np.float32)
_PWVEC = _WVEC + 2.0
_RVEC = np.array([_RL[l] for l in _COL_L], np.float32)
_HEADVEC = _COL_HEAD.astype(np.float32)

_ELX = np.zeros((2 * _L, 128), np.float32)
_ELY = np.zeros((2 * _L, 128), np.float32)
for _c in range(128):
    _ELX[2 * _COL_L[_c] + 0, _c] = 1.0
    _ELY[2 * _COL_L[_c] + 1, _c] = 1.0

_G = np.zeros((128, 128), np.float32)
for _i in range(128):
    for _j in range(128):
        if _COL_HEAD[_i] == _COL_HEAD[_j]:
            _G[_i, _j] = 1.0

_SINT = np.zeros((128, 128), np.float32)
for _c in range(4):
    for _hp in range(32):
        _SINT[_c * 32 + _hp, _hp * 4 + _c] = 1.0


def _tables_kernel(val_ref, wvt_ref, bv_ref, *out_refs):
    v = jnp.dot(val_ref[0], wvt_ref[:], preferred_element_type=jnp.float32, precision=lax.Precision.HIGHEST)
    v = v + bv_ref[:]
    start = 0
    for lvl, (hh, ww) in enumerate(_SPATIAL):
        vl = v[start:start + hh * ww].reshape(hh, ww, _D)
        start += hh * ww
        zc = jnp.zeros((hh, 1, _D), jnp.float32)
        vl = jnp.concatenate([zc, vl, zc], axis=1)
        zr = jnp.zeros((1, ww + 2, _D), jnp.float32)
        vl = jnp.concatenate([zr, vl, zr], axis=0)
        out_refs[lvl][0] = vl.reshape((hh + 2) * (ww + 2), _D)


def _build_tables(value, W_value, b_value):
    wvt = W_value.T
    bv = b_value.reshape(1, _D)
    out_shapes = [jax.ShapeDtypeStruct((_B, r, _D), jnp.float32)
                  for r in _RL]
    out_specs = [pl.BlockSpec((1, r, _D), lambda b: (b, 0, 0))
                 for r in _RL]
    tables = pl.pallas_call(
        _tables_kernel,
        grid=(_B,),
        in_specs=[
            pl.BlockSpec((1, _S, _D), lambda b: (b, 0, 0)),
            pl.BlockSpec((_D, _D), lambda b: (0, 0)),
            pl.BlockSpec((1, _D), lambda b: (0, 0)),
        ],
        out_specs=out_specs,
        out_shape=out_shapes,
    )(value, wvt, bv)
    return [t.reshape(_B * r * _H, _HD) for t, r in zip(tables, _RL)]


def _index_kernel(q_ref, ref_ref, woff_ref, boff_ref, wattn_ref, battn_ref,
                  g_ref, elx_ref, ely_ref, sint_ref, cvec_ref,
                  idx_ref, wts_ref):
    i = pl.program_id(0)
    qb = q_ref[:]
    off = jnp.dot(qb, woff_ref[:], preferred_element_type=jnp.float32, precision=lax.Precision.HIGHEST)
    off = off + boff_ref[:]
    ox = off[:, :128]
    oy = off[:, 128:]
    logit = jnp.dot(qb, wattn_ref[:], preferred_element_type=jnp.float32, precision=lax.Precision.HIGHEST)
    logit = logit + battn_ref[:]
    e = jnp.exp(logit)
    ssum = jnp.dot(e, g_ref[:], preferred_element_type=jnp.float32, precision=lax.Precision.HIGHEST)
    aw = e / ssum

    rx = jnp.dot(ref_ref[:], elx_ref[:],
                 preferred_element_type=jnp.float32, precision=lax.Precision.HIGHEST)
    ry = jnp.dot(ref_ref[:], ely_ref[:],
                 preferred_element_type=jnp.float32, precision=lax.Precision.HIGHEST)

    cvec = cvec_ref[:]
    wv = cvec[0:1, :]
    hv = cvec[1:2, :]
    gx = rx * wv + ox - 0.5
    gy = ry * hv + oy - 0.5
    x0 = jnp.floor(gx)
    y0 = jnp.floor(gy)
    wx1 = gx - x0
    wx0 = 1.0 - wx1
    wy1 = gy - y0
    wy0 = 1.0 - wy1
    px = jnp.clip(x0, -1.0, wv - 1.0)
    py = jnp.clip(y0, -1.0, hv - 1.0)
    vx = (x0 == px).astype(jnp.float32)
    vy = (y0 == py).astype(jnp.float32)
    wx0 = wx0 * vx
    wx1 = wx1 * vx
    wy0 = wy0 * vy
    wy1 = wy1 * vy

    wts_ref[:] = jnp.stack(
        [aw * wy0 * wx0, aw * wy0 * wx1, aw * wy1 * wx0, aw * wy1 * wx1],
        axis=1)

    rowf = (jnp.float32(i * _QBLK)
            + lax.broadcasted_iota(jnp.int32, (_QBLK, 128), 0
                                   ).astype(jnp.float32))
    bidx = jnp.floor(rowf / jnp.float32(_Q))
    pwv = cvec[2:3, :]
    base = (bidx * 8.0 * cvec[3:4, :]
            + ((py + 1.0) * pwv + (px + 1.0)) * 8.0 + cvec[4:5, :])
    sint = sint_ref[:]
    levels = []
    for lvl in range(_L):
        sl = slice(lvl * 32, (lvl + 1) * 32)
        pw8 = pwv[:, sl] * 8.0
        cat = jnp.concatenate(
            [base[:, sl], base[:, sl] + 8.0,
             base[:, sl] + pw8, base[:, sl] + pw8 + 8.0],
            axis=1)
        levels.append(jnp.dot(cat, sint,
                              preferred_element_type=jnp.float32, precision=lax.Precision.HIGHEST))
    idx_ref[:] = jnp.stack(levels, axis=1).astype(jnp.int32)


def _build_index(query, reference_points, W_off, b_off, W_attn, b_attn):
    perm_off = np.zeros(2 * _L * _H * _P, np.int64)
    for hd in range(_H):
        for lvl in range(_L):
            for p in range(_P):
                for xy in range(2):
                    src = ((hd * _L + lvl) * _P + p) * 2 + xy
                    dst = xy * 128 + lvl * 32 + hd * 4 + p
                    perm_off[dst] = src
    perm_attn = np.zeros(_L * _H * _P, np.int64)
    for hd in range(_H):
        for lvl in range(_L):
            for p in range(_P):
                src = (hd * _L + lvl) * _P + p
                dst = lvl * 32 + hd * 4 + p
                perm_attn[dst] = src
    woff_t = W_off[perm_off].T
    boff = b_off[perm_off].reshape(1, 256)
    wattn_t = W_attn[perm_attn].T
    battn = b_attn[perm_attn].reshape(1, 128)
    qf = query.reshape(_NQ, _D)
    rf = reference_points.reshape(_NQ, 2 * _L)
    cvec = np.zeros((8, 128), np.float32)
    cvec[0], cvec[1], cvec[2] = _WVEC, _HVEC, _PWVEC
    cvec[3], cvec[4] = _RVEC, _HEADVEC
    return pl.pallas_call(
        _index_kernel,
        grid=(_NBLK,),
        in_specs=[
            pl.BlockSpec((_QBLK, _D), lambda i: (i, 0)),
            pl.BlockSpec((_QBLK, 2 * _L), lambda i: (i, 0)),
            pl.BlockSpec((_D, _D), lambda i: (0, 0)),
            pl.BlockSpec((1, _D), lambda i: (0, 0)),
            pl.BlockSpec((_D, 128), lambda i: (0, 0)),
            pl.BlockSpec((1, 128), lambda i: (0, 0)),
            pl.BlockSpec((128, 128), lambda i: (0, 0)),
            pl.BlockSpec((2 * _L, 128), lambda i: (0, 0)),
            pl.BlockSpec((2 * _L, 128), lambda i: (0, 0)),
            pl.BlockSpec((128, 128), lambda i: (0, 0)),
            pl.BlockSpec((8, 128), lambda i: (0, 0)),
        ],
        out_specs=[
            pl.BlockSpec((_QBLK, _L, 128), lambda i: (i, 0, 0)),
            pl.BlockSpec((_QBLK, 4, 128), lambda i: (i, 0, 0)),
        ],
        out_shape=[
            jax.ShapeDtypeStruct((_NQ, _L, 128), jnp.int32),
            jax.ShapeDtypeStruct((_NQ, 4, 128), jnp.float32),
        ],
    )(qf, rf, woff_t, boff, wattn_t, battn,
      _G, _ELX, _ELY, _SINT, cvec)


_GRP = 9
_NGRP = _ROWS_PER_W // _GRP


def _sc_kernel(t0, t1, t2, t3, idx_hbm, wts_hbm, out_hbm,
               pkt_idx, pkt_wts, dst_v, out_v, sem_pkt, sem_g, sem_out):
    tabs = (t0, t1, t2, t3)
    wid = lax.axis_index("s") * 2 + lax.axis_index("c")
    qbase = wid * _ROWS_PER_W

    def issue_pkt(g, slot):
        pltpu.async_copy(idx_hbm.at[pl.ds(qbase + g * _GRP, _GRP)],
                         pkt_idx.at[slot], sem_pkt)
        pltpu.async_copy(wts_hbm.at[pl.ds(qbase + g * _GRP, _GRP)],
                         pkt_wts.at[slot], sem_pkt)

    def wait_pkt(slot):
        pltpu.make_async_copy(idx_hbm.at[pl.ds(qbase, _GRP)],
                              pkt_idx.at[slot], sem_pkt).wait()
        pltpu.make_async_copy(wts_hbm.at[pl.ds(qbase, _GRP)],
                              pkt_wts.at[slot], sem_pkt).wait()

    def issue_gathers(pslot, prow, gslot):
        for l in range(_L):
            pltpu.async_copy(tabs[l].at[pkt_idx.at[pslot, prow, l]],
                             dst_v.at[gslot, l], sem_g)

    def wait_gathers(gslot):
        for l in range(_L):
            pltpu.make_async_copy(tabs[l].at[pkt_idx.at[0, 0, l]],
                                  dst_v.at[gslot, l], sem_g).wait()

    def issue_out(g, oslot):
        pltpu.async_copy(out_v.at[oslot],
                         out_hbm.at[pl.ds((qbase + g * _GRP) * _H,
                                          _GRP * _H)], sem_out)

    def wait_out(oslot):
        pltpu.make_async_copy(out_v.at[oslot],
                              out_hbm.at[pl.ds(qbase * _H, _GRP * _H)],
                              sem_out).wait()

    issue_pkt(0, 0)
    wait_pkt(0)
    issue_pkt(1, 1)
    issue_gathers(0, 0, 0)

    def chunk(i, _):
        g = lax.div(i, _GRP)
        r9 = lax.rem(i, _GRP)
        gs = lax.rem(i, 2)
        ps = lax.rem(g, 3)
        oslot = lax.rem(g, 2)

        @pl.when(r9 == 0)
        def _():
            @pl.when(g + 2 < _NGRP)
            def _():
                issue_pkt(g + 2, lax.rem(g + 2, 3))

            @pl.when(g + 1 < _NGRP)
            def _():
                wait_pkt(lax.rem(g + 1, 3))

            @pl.when(g >= 2)
            def _():
                wait_out(oslot)

        n = i + 1

        @pl.when(n < _ROWS_PER_W)
        def _():
            issue_gathers(lax.rem(lax.div(n, _GRP), 3), lax.rem(n, _GRP),
                          1 - gs)

        wait_gathers(gs)

        wvecs = {}
        for lvl in range(_L):
            for c in range(4):
                for hg in range(2):
                    wvecs[(lvl, c, hg)] = pkt_wts[
                        ps, r9, c, pl.ds(lvl * 32 + hg * 16, 16)]
        for hd in range(_H):
            hg, j = hd // 4, hd % 4
            acc0 = jnp.zeros((16,), jnp.float32)
            acc1 = jnp.zeros((16,), jnp.float32)
            for lvl in range(_L):
                for p in range(_P):
                    for c in range(4):
                        r = hd * 16 + p * 4 + c
                        w = wvecs[(lvl, c, hg)][j * 4 + p]
                        wb = jnp.full((16,), w, jnp.float32)
                        acc0 = acc0 + wb * dst_v[gs, lvl, r, pl.ds(0, 16)]
                        acc1 = acc1 + wb * dst_v[gs, lvl, r, pl.ds(16, 16)]
            out_v[oslot, r9 * _H + hd, pl.ds(0, 16)] = acc0
            out_v[oslot, r9 * _H + hd, pl.ds(16, 16)] = acc1

        @pl.when(r9 == _GRP - 1)
        def _():
            issue_out(g, oslot)

        return 0

    lax.fori_loop(0, _ROWS_PER_W, chunk, 0)
    wait_out(lax.rem(_NGRP - 2, 2))
    wait_out(lax.rem(_NGRP - 1, 2))


def _sc_gather(tables, idx, wts):
    mesh = plsc.VectorSubcoreMesh(core_axis_name="c", subcore_axis_name="s")
    run = pl.kernel(
        _sc_kernel,
        out_type=jax.ShapeDtypeStruct((_N, _HD), jnp.float32),
        mesh=mesh,
        scratch_types=[
            pltpu.VMEM((3, _GRP, _L, 128), jnp.int32),
            pltpu.VMEM((3, _GRP, 4, 128), jnp.float32),
            pltpu.VMEM((2, _L, 128, _HD), jnp.float32),
            pltpu.VMEM((2, _GRP * _H, _HD), jnp.float32),
            pltpu.SemaphoreType.DMA,
            pltpu.SemaphoreType.DMA,
            pltpu.SemaphoreType.DMA,
        ],
        compiler_params=pltpu.CompilerParams(use_tc_tiling_on_sc=False),
    )
    return run(*tables, idx, wts)


def _proj_kernel(x_ref, w_ref, b_ref, o_ref):
    o_ref[:] = (jnp.dot(x_ref[:], w_ref[:], preferred_element_type=jnp.float32, precision=lax.Precision.HIGHEST)
                + b_ref[:])


def _out_proj(sampled, W_out, b_out):
    x = sampled.reshape(_NQ, _D)
    out = pl.pallas_call(
        _proj_kernel,
        grid=(_NBLK,),
        in_specs=[
            pl.BlockSpec((_QBLK, _D), lambda i: (i, 0)),
            pl.BlockSpec((_D, _D), lambda i: (0, 0)),
            pl.BlockSpec((1, _D), lambda i: (0, 0)),
        ],
        out_specs=pl.BlockSpec((_QBLK, _D), lambda i: (i, 0)),
        out_shape=jax.ShapeDtypeStruct((_NQ, _D), jnp.float32),
    )(x, W_out.T, b_out.reshape(1, _D))
    return out.reshape(_B, _Q, _D)


def kernel(query, reference_points, value, spatial_shapes, level_start_index,
           W_value, b_value, W_off, b_off, W_attn, b_attn, W_out, b_out):
    del spatial_shapes, level_start_index
    tables = _build_tables(value, W_value, b_value)
    idx, wts = _build_index(query, reference_points, W_off, b_off,
                            W_attn, b_attn)
    sampled = _sc_gather(tables, idx, wts)
    return _out_proj(sampled, W_out, b_out)


# speedup vs baseline: 95.3822x; 1.0578x over previous
import functools

import numpy as np
import jax
import jax.numpy as jnp
from jax import lax
from jax.experimental import pallas as pl
from jax.experimental.pallas import tpu as pltpu
from jax.experimental.pallas import tpu_sc as plsc

_SPATIAL = [(64, 64), (32, 32), (16, 16), (8, 8)]
_B, _Q, _D = 8, 900, 256
_H, _HD = 8, 32
_L, _P = 4, 4
_S = sum(h * w for h, w in _SPATIAL)
_NQ = _B * _Q
_N = _NQ * _H
_QBLK = 400
_NBLK = _NQ // _QBLK
_RL = [(h + 2) * (w + 2) for h, w in _SPATIAL]
_NW = 32
_ROWS_PER_W = _NQ // _NW

_COL_L = np.repeat(np.arange(_L), 32)
_COL_HEAD = np.tile(np.repeat(np.arange(_H), _P), _L)
_WVEC = np.array([_SPATIAL[l][1] for l in _COL_L], np.float32)
_HVEC = np.array([_SPATIAL[l][0] for l in _COL_L], np.float32)
_PWVEC = _WVEC + 2.0
_RVEC = np.array([_RL[l] for l in _COL_L], np.float32)
_HEADVEC = _COL_HEAD.astype(np.float32)

_ELX = np.zeros((2 * _L, 128), np.float32)
_ELY = np.zeros((2 * _L, 128), np.float32)
for _c in range(128):
    _ELX[2 * _COL_L[_c] + 0, _c] = 1.0
    _ELY[2 * _COL_L[_c] + 1, _c] = 1.0

_G = np.zeros((128, 128), np.float32)
for _i in range(128):
    for _j in range(128):
        if _COL_HEAD[_i] == _COL_HEAD[_j]:
            _G[_i, _j] = 1.0

_SINT = np.zeros((128, 128), np.float32)
for _c in range(4):
    for _hp in range(32):
        _SINT[_c * 32 + _hp, _hp * 4 + _c] = 1.0


def _tables_kernel(val_ref, wvt_ref, bv_ref, *out_refs):
    v = jnp.dot(val_ref[0], wvt_ref[:], preferred_element_type=jnp.float32)
    v = v + bv_ref[:]
    start = 0
    for lvl, (hh, ww) in enumerate(_SPATIAL):
        vl = v[start:start + hh * ww].reshape(hh, ww, _D)
        start += hh * ww
        zc = jnp.zeros((hh, 1, _D), jnp.float32)
        vl = jnp.concatenate([zc, vl, zc], axis=1)
        zr = jnp.zeros((1, ww + 2, _D), jnp.float32)
        vl = jnp.concatenate([zr, vl, zr], axis=0)
        out_refs[lvl][0] = vl.reshape((hh + 2) * (ww + 2), _D)


def _build_tables(value, W_value, b_value):
    wvt = W_value.T
    bv = b_value.reshape(1, _D)
    out_shapes = [jax.ShapeDtypeStruct((_B, r, _D), jnp.float32)
                  for r in _RL]
    out_specs = [pl.BlockSpec((1, r, _D), lambda b: (b, 0, 0))
                 for r in _RL]
    tables = pl.pallas_call(
        _tables_kernel,
        grid=(_B,),
        in_specs=[
            pl.BlockSpec((1, _S, _D), lambda b: (b, 0, 0)),
            pl.BlockSpec((_D, _D), lambda b: (0, 0)),
            pl.BlockSpec((1, _D), lambda b: (0, 0)),
        ],
        out_specs=out_specs,
        out_shape=out_shapes,
    )(value, wvt, bv)
    return [t.reshape(_B * r * _H, _HD) for t, r in zip(tables, _RL)]


def _index_kernel(q_ref, ref_ref, woff_ref, boff_ref, wattn_ref, battn_ref,
                  g_ref, elx_ref, ely_ref, sint_ref, cvec_ref,
                  idx_ref, wts_ref):
    i = pl.program_id(0)
    qb = q_ref[:]
    off = jnp.dot(qb, woff_ref[:], preferred_element_type=jnp.float32)
    off = off + boff_ref[:]
    ox = off[:, :128]
    oy = off[:, 128:]
    logit = jnp.dot(qb, wattn_ref[:], preferred_element_type=jnp.float32)
    logit = logit + battn_ref[:]
    e = jnp.exp(logit)
    ssum = jnp.dot(e, g_ref[:], preferred_element_type=jnp.float32, precision=lax.Precision.HIGHEST)
    aw = e / ssum

    rx = jnp.dot(ref_ref[:], elx_ref[:],
                 preferred_element_type=jnp.float32, precision=lax.Precision.HIGHEST)
    ry = jnp.dot(ref_ref[:], ely_ref[:],
                 preferred_element_type=jnp.float32, precision=lax.Precision.HIGHEST)

    cvec = cvec_ref[:]
    wv = cvec[0:1, :]
    hv = cvec[1:2, :]
    gx = rx * wv + ox - 0.5
    gy = ry * hv + oy - 0.5
    x0 = jnp.floor(gx)
    y0 = jnp.floor(gy)
    wx1 = gx - x0
    wx0 = 1.0 - wx1
    wy1 = gy - y0
    wy0 = 1.0 - wy1
    px = jnp.clip(x0, -1.0, wv - 1.0)
    py = jnp.clip(y0, -1.0, hv - 1.0)
    vx = (x0 == px).astype(jnp.float32)
    vy = (y0 == py).astype(jnp.float32)
    wx0 = wx0 * vx
    wx1 = wx1 * vx
    wy0 = wy0 * vy
    wy1 = wy1 * vy

    wts_ref[:] = jnp.stack(
        [aw * wy0 * wx0, aw * wy0 * wx1, aw * wy1 * wx0, aw * wy1 * wx1],
        axis=1)

    rowf = (jnp.float32(i * _QBLK)
            + lax.broadcasted_iota(jnp.int32, (_QBLK, 128), 0
                                   ).astype(jnp.float32))
    bidx = jnp.floor(rowf / jnp.float32(_Q))
    pwv = cvec[2:3, :]
    base = (bidx * 8.0 * cvec[3:4, :]
            + ((py + 1.0) * pwv + (px + 1.0)) * 8.0 + cvec[4:5, :])
    sint = sint_ref[:]
    levels = []
    for lvl in range(_L):
        sl = slice(lvl * 32, (lvl + 1) * 32)
        pw8 = pwv[:, sl] * 8.0
        cat = jnp.concatenate(
            [base[:, sl], base[:, sl] + 8.0,
             base[:, sl] + pw8, base[:, sl] + pw8 + 8.0],
            axis=1)
        levels.append(jnp.dot(cat, sint,
                              preferred_element_type=jnp.float32, precision=lax.Precision.HIGHEST))
    idx_ref[:] = jnp.stack(levels, axis=1).astype(jnp.int32)


def _build_index(query, reference_points, W_off, b_off, W_attn, b_attn):
    perm_off = np.zeros(2 * _L * _H * _P, np.int64)
    for hd in range(_H):
        for lvl in range(_L):
            for p in range(_P):
                for xy in range(2):
                    src = ((hd * _L + lvl) * _P + p) * 2 + xy
                    dst = xy * 128 + lvl * 32 + hd * 4 + p
                    perm_off[dst] = src
    perm_attn = np.zeros(_L * _H * _P, np.int64)
    for hd in range(_H):
        for lvl in range(_L):
            for p in range(_P):
                src = (hd * _L + lvl) * _P + p
                dst = lvl * 32 + hd * 4 + p
                perm_attn[dst] = src
    woff_t = W_off[perm_off].T
    boff = b_off[perm_off].reshape(1, 256)
    wattn_t = W_attn[perm_attn].T
    battn = b_attn[perm_attn].reshape(1, 128)
    qf = query.reshape(_NQ, _D)
    rf = reference_points.reshape(_NQ, 2 * _L)
    cvec = np.zeros((8, 128), np.float32)
    cvec[0], cvec[1], cvec[2] = _WVEC, _HVEC, _PWVEC
    cvec[3], cvec[4] = _RVEC, _HEADVEC
    return pl.pallas_call(
        _index_kernel,
        grid=(_NBLK,),
        in_specs=[
            pl.BlockSpec((_QBLK, _D), lambda i: (i, 0)),
            pl.BlockSpec((_QBLK, 2 * _L), lambda i: (i, 0)),
            pl.BlockSpec((_D, _D), lambda i: (0, 0)),
            pl.BlockSpec((1, _D), lambda i: (0, 0)),
            pl.BlockSpec((_D, 128), lambda i: (0, 0)),
            pl.BlockSpec((1, 128), lambda i: (0, 0)),
            pl.BlockSpec((128, 128), lambda i: (0, 0)),
            pl.BlockSpec((2 * _L, 128), lambda i: (0, 0)),
            pl.BlockSpec((2 * _L, 128), lambda i: (0, 0)),
            pl.BlockSpec((128, 128), lambda i: (0, 0)),
            pl.BlockSpec((8, 128), lambda i: (0, 0)),
        ],
        out_specs=[
            pl.BlockSpec((_QBLK, _L, 128), lambda i: (i, 0, 0)),
            pl.BlockSpec((_QBLK, 4, 128), lambda i: (i, 0, 0)),
        ],
        out_shape=[
            jax.ShapeDtypeStruct((_NQ, _L, 128), jnp.int32),
            jax.ShapeDtypeStruct((_NQ, 4, 128), jnp.float32),
        ],
    )(qf, rf, woff_t, boff, wattn_t, battn,
      _G, _ELX, _ELY, _SINT, cvec)


_GRP = 9
_NGRP = _ROWS_PER_W // _GRP


def _sc_kernel(t0, t1, t2, t3, idx_hbm, wts_hbm, out_hbm,
               pkt_idx, pkt_wts, dst_v, out_v, sem_pkt, sem_g, sem_out):
    tabs = (t0, t1, t2, t3)
    wid = lax.axis_index("s") * 2 + lax.axis_index("c")
    qbase = wid * _ROWS_PER_W

    def issue_pkt(g, slot):
        pltpu.async_copy(idx_hbm.at[pl.ds(qbase + g * _GRP, _GRP)],
                         pkt_idx.at[slot], sem_pkt)
        pltpu.async_copy(wts_hbm.at[pl.ds(qbase + g * _GRP, _GRP)],
                         pkt_wts.at[slot], sem_pkt)

    def wait_pkt(slot):
        pltpu.make_async_copy(idx_hbm.at[pl.ds(qbase, _GRP)],
                              pkt_idx.at[slot], sem_pkt).wait()
        pltpu.make_async_copy(wts_hbm.at[pl.ds(qbase, _GRP)],
                              pkt_wts.at[slot], sem_pkt).wait()

    def issue_gathers(pslot, prow, gslot):
        for l in range(_L):
            pltpu.async_copy(tabs[l].at[pkt_idx.at[pslot, prow, l]],
                             dst_v.at[gslot, l], sem_g)

    def wait_gathers(gslot):
        for l in range(_L):
            pltpu.make_async_copy(tabs[l].at[pkt_idx.at[0, 0, l]],
                                  dst_v.at[gslot, l], sem_g).wait()

    def issue_out(g, oslot):
        pltpu.async_copy(out_v.at[oslot],
                         out_hbm.at[pl.ds((qbase + g * _GRP) * _H,
                                          _GRP * _H)], sem_out)

    def wait_out(oslot):
        pltpu.make_async_copy(out_v.at[oslot],
                              out_hbm.at[pl.ds(qbase * _H, _GRP * _H)],
                              sem_out).wait()

    issue_pkt(0, 0)
    wait_pkt(0)
    issue_pkt(1, 1)
    issue_gathers(0, 0, 0)

    def chunk(i, _):
        g = lax.div(i, _GRP)
        r9 = lax.rem(i, _GRP)
        gs = lax.rem(i, 2)
        ps = lax.rem(g, 3)
        oslot = lax.rem(g, 2)

        @pl.when(r9 == 0)
        def _():
            @pl.when(g + 2 < _NGRP)
            def _():
                issue_pkt(g + 2, lax.rem(g + 2, 3))

            @pl.when(g + 1 < _NGRP)
            def _():
                wait_pkt(lax.rem(g + 1, 3))

            @pl.when(g >= 2)
            def _():
                wait_out(oslot)

        n = i + 1

        @pl.when(n < _ROWS_PER_W)
        def _():
            issue_gathers(lax.rem(lax.div(n, _GRP), 3), lax.rem(n, _GRP),
                          1 - gs)

        wait_gathers(gs)

        wvecs = {}
        for lvl in range(_L):
            for c in range(4):
                for hg in range(2):
                    wvecs[(lvl, c, hg)] = pkt_wts[
                        ps, r9, c, pl.ds(lvl * 32 + hg * 16, 16)]
        for hd in range(_H):
            hg, j = hd // 4, hd % 4
            acc0 = jnp.zeros((16,), jnp.float32)
            acc1 = jnp.zeros((16,), jnp.float32)
            for lvl in range(_L):
                for p in range(_P):
                    for c in range(4):
                        r = hd * 16 + p * 4 + c
                        w = wvecs[(lvl, c, hg)][j * 4 + p]
                        wb = jnp.full((16,), w, jnp.float32)
                        acc0 = acc0 + wb * dst_v[gs, lvl, r, pl.ds(0, 16)]
                        acc1 = acc1 + wb * dst_v[gs, lvl, r, pl.ds(16, 16)]
            out_v[oslot, r9 * _H + hd, pl.ds(0, 16)] = acc0
            out_v[oslot, r9 * _H + hd, pl.ds(16, 16)] = acc1

        @pl.when(r9 == _GRP - 1)
        def _():
            issue_out(g, oslot)

        return 0

    lax.fori_loop(0, _ROWS_PER_W, chunk, 0)
    wait_out(lax.rem(_NGRP - 2, 2))
    wait_out(lax.rem(_NGRP - 1, 2))


def _sc_gather(tables, idx, wts):
    mesh = plsc.VectorSubcoreMesh(core_axis_name="c", subcore_axis_name="s")
    run = pl.kernel(
        _sc_kernel,
        out_type=jax.ShapeDtypeStruct((_N, _HD), jnp.float32),
        mesh=mesh,
        scratch_types=[
            pltpu.VMEM((3, _GRP, _L, 128), jnp.int32),
            pltpu.VMEM((3, _GRP, 4, 128), jnp.float32),
            pltpu.VMEM((2, _L, 128, _HD), jnp.float32),
            pltpu.VMEM((2, _GRP * _H, _HD), jnp.float32),
            pltpu.SemaphoreType.DMA,
            pltpu.SemaphoreType.DMA,
            pltpu.SemaphoreType.DMA,
        ],
        compiler_params=pltpu.CompilerParams(use_tc_tiling_on_sc=False),
    )
    return run(*tables, idx, wts)


def _proj_kernel(x_ref, w_ref, b_ref, o_ref):
    o_ref[:] = (jnp.dot(x_ref[:], w_ref[:], preferred_element_type=jnp.float32, precision=lax.Precision.HIGHEST)
                + b_ref[:])


def _out_proj(sampled, W_out, b_out):
    x = sampled.reshape(_NQ, _D)
    out = pl.pallas_call(
        _proj_kernel,
        grid=(_NBLK,),
        in_specs=[
            pl.BlockSpec((_QBLK, _D), lambda i: (i, 0)),
            pl.BlockSpec((_D, _D), lambda i: (0, 0)),
            pl.BlockSpec((1, _D), lambda i: (0, 0)),
        ],
        out_specs=pl.BlockSpec((_QBLK, _D), lambda i: (i, 0)),
        out_shape=jax.ShapeDtypeStruct((_NQ, _D), jnp.float32),
    )(x, W_out.T, b_out.reshape(1, _D))
    return out.reshape(_B, _Q, _D)


def kernel(query, reference_points, value, spatial_shapes, level_start_index,
           W_value, b_value, W_off, b_off, W_attn, b_attn, W_out, b_out):
    del spatial_shapes, level_start_index
    tables = _build_tables(value, W_value, b_value)
    idx, wts = _build_index(query, reference_points, W_off, b_off,
                            W_attn, b_attn)
    sampled = _sc_gather(tables, idx, wts)
    return _out_proj(sampled, W_out, b_out)


# speedup vs baseline: 107.4556x; 1.1266x over previous
import functools

import numpy as np
import jax
import jax.numpy as jnp
from jax import lax
from jax.experimental import pallas as pl
from jax.experimental.pallas import tpu as pltpu
from jax.experimental.pallas import tpu_sc as plsc

_SPATIAL = [(64, 64), (32, 32), (16, 16), (8, 8)]
_B, _Q, _D = 8, 900, 256
_H, _HD = 8, 32
_L, _P = 4, 4
_S = sum(h * w for h, w in _SPATIAL)
_NQ = _B * _Q
_N = _NQ * _H
_QBLK = 400
_NBLK = _NQ // _QBLK
_RL = [(h + 2) * (w + 2) for h, w in _SPATIAL]
_NW = 32
_ROWS_PER_W = _NQ // _NW

_COL_L = np.repeat(np.arange(_L), 32)
_COL_HEAD = np.tile(np.repeat(np.arange(_H), _P), _L)
_WVEC = np.array([_SPATIAL[l][1] for l in _COL_L], np.float32)
_HVEC = np.array([_SPATIAL[l][0] for l in _COL_L], np.float32)
_PWVEC = _WVEC + 2.0
_RVEC = np.array([_RL[l] for l in _COL_L], np.float32)
_HEADVEC = _COL_HEAD.astype(np.float32)

_ELX = np.zeros((2 * _L, 128), np.float32)
_ELY = np.zeros((2 * _L, 128), np.float32)
for _c in range(128):
    _ELX[2 * _COL_L[_c] + 0, _c] = 1.0
    _ELY[2 * _COL_L[_c] + 1, _c] = 1.0

_G = np.zeros((128, 128), np.float32)
for _i in range(128):
    for _j in range(128):
        if _COL_HEAD[_i] == _COL_HEAD[_j]:
            _G[_i, _j] = 1.0

_SINT = np.zeros((128, 128), np.float32)
for _c in range(4):
    for _hp in range(32):
        _SINT[_c * 32 + _hp, _hp * 4 + _c] = 1.0


def _tables_kernel(val_ref, wvt_ref, bv_ref, *out_refs):
    v = jnp.dot(val_ref[0], wvt_ref[:], preferred_element_type=jnp.float32)
    v = v + bv_ref[:]
    start = 0
    for lvl, (hh, ww) in enumerate(_SPATIAL):
        vl = v[start:start + hh * ww].reshape(hh, ww, _D)
        start += hh * ww
        zc = jnp.zeros((hh, 1, _D), jnp.float32)
        vl = jnp.concatenate([zc, vl, zc], axis=1)
        zr = jnp.zeros((1, ww + 2, _D), jnp.float32)
        vl = jnp.concatenate([zr, vl, zr], axis=0)
        out_refs[lvl][:] = vl.reshape((hh + 2) * (ww + 2) * 2, 128)


def _build_tables(value, W_value, b_value):
    wvt = W_value.T
    bv = b_value.reshape(1, _D)
    out_shapes = [jax.ShapeDtypeStruct((_B * r * 2, 128), jnp.float32)
                  for r in _RL]
    out_specs = [pl.BlockSpec((r * 2, 128), lambda b: (b, 0))
                 for r in _RL]
    tables = pl.pallas_call(
        _tables_kernel,
        grid=(_B,),
        in_specs=[
            pl.BlockSpec((1, _S, _D), lambda b: (b, 0, 0)),
            pl.BlockSpec((_D, _D), lambda b: (0, 0)),
            pl.BlockSpec((1, _D), lambda b: (0, 0)),
        ],
        out_specs=out_specs,
        out_shape=out_shapes,
    )(value, wvt, bv)
    return [t.reshape(_B * r * _H, _HD) for t, r in zip(tables, _RL)]


def _index_kernel(q_ref, ref_ref, woff_ref, boff_ref, wattn_ref, battn_ref,
                  g_ref, elx_ref, ely_ref, sint_ref, cvec_ref,
                  idx_ref, wts_ref):
    i = pl.program_id(0)
    qb = q_ref[:]
    off = jnp.dot(qb, woff_ref[:], preferred_element_type=jnp.float32)
    off = off + boff_ref[:]
    ox = off[:, :128]
    oy = off[:, 128:]
    logit = jnp.dot(qb, wattn_ref[:], preferred_element_type=jnp.float32)
    logit = logit + battn_ref[:]
    e = jnp.exp(logit)
    ssum = jnp.dot(e, g_ref[:], preferred_element_type=jnp.float32, precision=lax.Precision.HIGHEST)
    aw = e / ssum

    rx = jnp.dot(ref_ref[:], elx_ref[:],
                 preferred_element_type=jnp.float32, precision=lax.Precision.HIGHEST)
    ry = jnp.dot(ref_ref[:], ely_ref[:],
                 preferred_element_type=jnp.float32, precision=lax.Precision.HIGHEST)

    cvec = cvec_ref[:]
    wv = cvec[0:1, :]
    hv = cvec[1:2, :]
    gx = rx * wv + ox - 0.5
    gy = ry * hv + oy - 0.5
    x0 = jnp.floor(gx)
    y0 = jnp.floor(gy)
    wx1 = gx - x0
    wx0 = 1.0 - wx1
    wy1 = gy - y0
    wy0 = 1.0 - wy1
    px = jnp.clip(x0, -1.0, wv - 1.0)
    py = jnp.clip(y0, -1.0, hv - 1.0)
    vx = (x0 == px).astype(jnp.float32)
    vy = (y0 == py).astype(jnp.float32)
    wx0 = wx0 * vx
    wx1 = wx1 * vx
    wy0 = wy0 * vy
    wy1 = wy1 * vy

    wts_ref[:] = jnp.stack(
        [aw * wy0 * wx0, aw * wy0 * wx1, aw * wy1 * wx0, aw * wy1 * wx1],
        axis=1).reshape(4 * _QBLK, 128)

    rowf = (jnp.float32(i * _QBLK)
            + lax.broadcasted_iota(jnp.int32, (_QBLK, 128), 0
                                   ).astype(jnp.float32))
    bidx = jnp.floor(rowf / jnp.float32(_Q))
    pwv = cvec[2:3, :]
    base = (bidx * 8.0 * cvec[3:4, :]
            + ((py + 1.0) * pwv + (px + 1.0)) * 8.0 + cvec[4:5, :])
    sint = sint_ref[:]
    levels = []
    for lvl in range(_L):
        sl = slice(lvl * 32, (lvl + 1) * 32)
        pw8 = pwv[:, sl] * 8.0
        cat = jnp.concatenate(
            [base[:, sl], base[:, sl] + 8.0,
             base[:, sl] + pw8, base[:, sl] + pw8 + 8.0],
            axis=1)
        levels.append(jnp.dot(cat, sint,
                              preferred_element_type=jnp.float32, precision=lax.Precision.HIGHEST))
    idx_ref[:] = jnp.stack(levels, axis=1).astype(jnp.int32).reshape(
        4 * _QBLK, 128)


def _build_index(query, reference_points, W_off, b_off, W_attn, b_attn):
    perm_off = np.zeros(2 * _L * _H * _P, np.int64)
    for hd in range(_H):
        for lvl in range(_L):
            for p in range(_P):
                for xy in range(2):
                    src = ((hd * _L + lvl) * _P + p) * 2 + xy
                    dst = xy * 128 + lvl * 32 + hd * 4 + p
                    perm_off[dst] = src
    perm_attn = np.zeros(_L * _H * _P, np.int64)
    for hd in range(_H):
        for lvl in range(_L):
            for p in range(_P):
                src = (hd * _L + lvl) * _P + p
                dst = lvl * 32 + hd * 4 + p
                perm_attn[dst] = src
    woff_t = W_off[perm_off].T
    boff = b_off[perm_off].reshape(1, 256)
    wattn_t = W_attn[perm_attn].T
    battn = b_attn[perm_attn].reshape(1, 128)
    qf = query.reshape(_NQ, _D)
    rf = reference_points.reshape(_NQ, 2 * _L)
    cvec = np.zeros((8, 128), np.float32)
    cvec[0], cvec[1], cvec[2] = _WVEC, _HVEC, _PWVEC
    cvec[3], cvec[4] = _RVEC, _HEADVEC
    return pl.pallas_call(
        _index_kernel,
        grid=(_NBLK,),
        in_specs=[
            pl.BlockSpec((_QBLK, _D), lambda i: (i, 0)),
            pl.BlockSpec((_QBLK, 2 * _L), lambda i: (i, 0)),
            pl.BlockSpec((_D, _D), lambda i: (0, 0)),
            pl.BlockSpec((1, _D), lambda i: (0, 0)),
            pl.BlockSpec((_D, 128), lambda i: (0, 0)),
            pl.BlockSpec((1, 128), lambda i: (0, 0)),
            pl.BlockSpec((128, 128), lambda i: (0, 0)),
            pl.BlockSpec((2 * _L, 128), lambda i: (0, 0)),
            pl.BlockSpec((2 * _L, 128), lambda i: (0, 0)),
            pl.BlockSpec((128, 128), lambda i: (0, 0)),
            pl.BlockSpec((8, 128), lambda i: (0, 0)),
        ],
        out_specs=[
            pl.BlockSpec((_QBLK * _L, 128), lambda i: (i, 0)),
            pl.BlockSpec((_QBLK * 4, 128), lambda i: (i, 0)),
        ],
        out_shape=[
            jax.ShapeDtypeStruct((_NQ * _L, 128), jnp.int32),
            jax.ShapeDtypeStruct((_NQ * 4, 128), jnp.float32),
        ],
    )(qf, rf, woff_t, boff, wattn_t, battn,
      _G, _ELX, _ELY, _SINT, cvec)


_GRP = 9
_NGRP = _ROWS_PER_W // _GRP


def _sc_kernel(t0, t1, t2, t3, idx_hbm, wts_hbm, out_hbm,
               pkt_idx, pkt_wts, dst_v, out_v, sem_pkt, sem_g, sem_out):
    tabs = (t0, t1, t2, t3)
    wid = lax.axis_index("s") * 2 + lax.axis_index("c")
    qbase = wid * _ROWS_PER_W

    def issue_pkt(g, slot):
        pltpu.async_copy(idx_hbm.at[pl.ds(qbase + g * _GRP, _GRP)],
                         pkt_idx.at[slot], sem_pkt)
        pltpu.async_copy(wts_hbm.at[pl.ds(qbase + g * _GRP, _GRP)],
                         pkt_wts.at[slot], sem_pkt)

    def wait_pkt(slot):
        pltpu.make_async_copy(idx_hbm.at[pl.ds(qbase, _GRP)],
                              pkt_idx.at[slot], sem_pkt).wait()
        pltpu.make_async_copy(wts_hbm.at[pl.ds(qbase, _GRP)],
                              pkt_wts.at[slot], sem_pkt).wait()

    def issue_gathers(pslot, prow, gslot):
        for l in range(_L):
            pltpu.async_copy(tabs[l].at[pkt_idx.at[pslot, prow, l]],
                             dst_v.at[gslot, l], sem_g)

    def wait_gathers(gslot):
        for l in range(_L):
            pltpu.make_async_copy(tabs[l].at[pkt_idx.at[0, 0, l]],
                                  dst_v.at[gslot, l], sem_g).wait()

    def issue_out(g, oslot):
        pltpu.async_copy(out_v.at[oslot],
                         out_hbm.at[pl.ds((qbase + g * _GRP) * 2,
                                          _GRP * 2)], sem_out)

    def wait_out(oslot):
        pltpu.make_async_copy(out_v.at[oslot],
                              out_hbm.at[pl.ds(qbase * 2, _GRP * 2)],
                              sem_out).wait()

    issue_pkt(0, 0)
    wait_pkt(0)
    issue_pkt(1, 1)
    issue_gathers(0, 0, 0)

    def chunk(i, _):
        g = lax.div(i, _GRP)
        r9 = lax.rem(i, _GRP)
        gs = lax.rem(i, 2)
        ps = lax.rem(g, 3)
        oslot = lax.rem(g, 2)

        @pl.when(r9 == 0)
        def _():
            @pl.when(g + 2 < _NGRP)
            def _():
                issue_pkt(g + 2, lax.rem(g + 2, 3))

            @pl.when(g + 1 < _NGRP)
            def _():
                wait_pkt(lax.rem(g + 1, 3))

            @pl.when(g >= 2)
            def _():
                wait_out(oslot)

        n = i + 1

        @pl.when(n < _ROWS_PER_W)
        def _():
            issue_gathers(lax.rem(lax.div(n, _GRP), 3), lax.rem(n, _GRP),
                          1 - gs)

        wait_gathers(gs)

        wvecs = {}
        for lvl in range(_L):
            for c in range(4):
                for hg in range(2):
                    wvecs[(lvl, c, hg)] = pkt_wts[
                        ps, r9, c, pl.ds(lvl * 32 + hg * 16, 16)]
        for hd in range(_H):
            hg, j = hd // 4, hd % 4
            acc0 = jnp.zeros((16,), jnp.float32)
            acc1 = jnp.zeros((16,), jnp.float32)
            for lvl in range(_L):
                for p in range(_P):
                    for c in range(4):
                        r = hd * 16 + p * 4 + c
                        w = wvecs[(lvl, c, hg)][j * 4 + p]
                        wb = jnp.full((16,), w, jnp.float32)
                        acc0 = acc0 + wb * dst_v[gs, lvl, r, pl.ds(0, 16)]
                        acc1 = acc1 + wb * dst_v[gs, lvl, r, pl.ds(16, 16)]
            orow = r9 * 2 + hd // 4
            ocol = (hd % 4) * 32
            out_v[oslot, orow, pl.ds(ocol, 16)] = acc0
            out_v[oslot, orow, pl.ds(ocol + 16, 16)] = acc1

        @pl.when(r9 == _GRP - 1)
        def _():
            issue_out(g, oslot)

        return 0

    lax.fori_loop(0, _ROWS_PER_W, chunk, 0)
    wait_out(lax.rem(_NGRP - 2, 2))
    wait_out(lax.rem(_NGRP - 1, 2))


def _sc_gather(tables, idx, wts):
    mesh = plsc.VectorSubcoreMesh(core_axis_name="c", subcore_axis_name="s")
    run = pl.kernel(
        _sc_kernel,
        out_type=jax.ShapeDtypeStruct((_NQ * 2, 128), jnp.float32),
        mesh=mesh,
        scratch_types=[
            pltpu.VMEM((3, _GRP, _L, 128), jnp.int32),
            pltpu.VMEM((3, _GRP, 4, 128), jnp.float32),
            pltpu.VMEM((2, _L, 128, _HD), jnp.float32),
            pltpu.VMEM((2, _GRP * 2, 128), jnp.float32),
            pltpu.SemaphoreType.DMA,
            pltpu.SemaphoreType.DMA,
            pltpu.SemaphoreType.DMA,
        ],
        compiler_params=pltpu.CompilerParams(use_tc_tiling_on_sc=False),
    )
    return run(*tables, idx, wts)


def _proj_kernel(x_ref, w_ref, b_ref, o_ref):
    x = x_ref[:].reshape(_QBLK, _D)
    o_ref[:] = (jnp.dot(x, w_ref[:], preferred_element_type=jnp.float32, precision=lax.Precision.HIGHEST)
                + b_ref[:])


def _out_proj(sampled, W_out, b_out):
    x = sampled
    out = pl.pallas_call(
        _proj_kernel,
        grid=(_NBLK,),
        in_specs=[
            pl.BlockSpec((_QBLK * 2, 128), lambda i: (i, 0)),
            pl.BlockSpec((_D, _D), lambda i: (0, 0)),
            pl.BlockSpec((1, _D), lambda i: (0, 0)),
        ],
        out_specs=pl.BlockSpec((_QBLK, _D), lambda i: (i, 0)),
        out_shape=jax.ShapeDtypeStruct((_NQ, _D), jnp.float32),
    )(x, W_out.T, b_out.reshape(1, _D))
    return out.reshape(_B, _Q, _D)


def kernel(query, reference_points, value, spatial_shapes, level_start_index,
           W_value, b_value, W_off, b_off, W_attn, b_attn, W_out, b_out):
    del spatial_shapes, level_start_index
    tables = _build_tables(value, W_value, b_value)
    idx, wts = _build_index(query, reference_points, W_off, b_off,
                            W_attn, b_attn)
    idx = idx.reshape(_NQ, _L, 128)
    wts = wts.reshape(_NQ, 4, 128)
    sampled = _sc_gather(tables, idx, wts)
    return _out_proj(sampled, W_out, b_out)


# speedup vs baseline: 117.2067x; 1.0907x over previous
import functools

import numpy as np
import jax
import jax.numpy as jnp
from jax import lax
from jax.experimental import pallas as pl
from jax.experimental.pallas import tpu as pltpu
from jax.experimental.pallas import tpu_sc as plsc

_SPATIAL = [(64, 64), (32, 32), (16, 16), (8, 8)]
_B, _Q, _D = 8, 900, 256
_H, _HD = 8, 32
_L, _P = 4, 4
_S = sum(h * w for h, w in _SPATIAL)
_NQ = _B * _Q
_N = _NQ * _H
_QBLK = 400
_NBLK = _NQ // _QBLK
_RL = [(h + 2) * (w + 2) for h, w in _SPATIAL]
_NW = 32
_ROWS_PER_W = _NQ // _NW

_COL_L = np.repeat(np.arange(_L), 32)
_COL_HEAD = np.tile(np.repeat(np.arange(_H), _P), _L)
_WVEC = np.array([_SPATIAL[l][1] for l in _COL_L], np.float32)
_HVEC = np.array([_SPATIAL[l][0] for l in _COL_L], np.float32)
_PWVEC = _WVEC + 2.0
_RVEC = np.array([_RL[l] for l in _COL_L], np.float32)
_HEADVEC = _COL_HEAD.astype(np.float32)

_ELX = np.zeros((2 * _L, 128), np.float32)
_ELY = np.zeros((2 * _L, 128), np.float32)
for _c in range(128):
    _ELX[2 * _COL_L[_c] + 0, _c] = 1.0
    _ELY[2 * _COL_L[_c] + 1, _c] = 1.0

_G = np.zeros((128, 128), np.float32)
for _i in range(128):
    for _j in range(128):
        if _COL_HEAD[_i] == _COL_HEAD[_j]:
            _G[_i, _j] = 1.0

_SINT = np.zeros((128, 128), np.float32)
for _c in range(4):
    for _hp in range(32):
        _SINT[_c * 32 + _hp, _hp * 4 + _c] = 1.0


def _tables_kernel(val_ref, wvt_ref, bv_ref, *out_refs):
    v = jnp.dot(val_ref[0], wvt_ref[:], preferred_element_type=jnp.float32)
    v = v + bv_ref[:]
    start = 0
    for lvl, (hh, ww) in enumerate(_SPATIAL):
        vl = v[start:start + hh * ww].reshape(hh, ww, _D)
        start += hh * ww
        zc = jnp.zeros((hh, 1, _D), jnp.float32)
        vl = jnp.concatenate([zc, vl, zc], axis=1)
        zr = jnp.zeros((1, ww + 2, _D), jnp.float32)
        vl = jnp.concatenate([zr, vl, zr], axis=0)
        out_refs[lvl][:] = vl.reshape((hh + 2) * (ww + 2) * 2, 128)


def _build_tables(value, W_value, b_value):
    wvt = W_value.T
    bv = b_value.reshape(1, _D)
    out_shapes = [jax.ShapeDtypeStruct((_B * r * 2, 128), jnp.float32)
                  for r in _RL]
    out_specs = [pl.BlockSpec((r * 2, 128), lambda b: (b, 0))
                 for r in _RL]
    tables = pl.pallas_call(
        _tables_kernel,
        grid=(_B,),
        in_specs=[
            pl.BlockSpec((1, _S, _D), lambda b: (b, 0, 0)),
            pl.BlockSpec((_D, _D), lambda b: (0, 0)),
            pl.BlockSpec((1, _D), lambda b: (0, 0)),
        ],
        out_specs=out_specs,
        out_shape=out_shapes,
    )(value, wvt, bv)
    return [t.reshape(_B * r * _H, _HD) for t, r in zip(tables, _RL)]


def _index_kernel(q_ref, ref_ref, woff_ref, boff_ref, wattn_ref, battn_ref,
                  g_ref, elx_ref, ely_ref, cvec_ref,
                  idx_ref, wts_ref):
    i = pl.program_id(0)
    qb = q_ref[:]
    off = jnp.dot(qb, woff_ref[:], preferred_element_type=jnp.float32)
    off = off + boff_ref[:]
    ox = off[:, :128]
    oy = off[:, 128:]
    logit = jnp.dot(qb, wattn_ref[:], preferred_element_type=jnp.float32)
    logit = logit + battn_ref[:]
    e = jnp.exp(logit)
    ssum = jnp.dot(e, g_ref[:], preferred_element_type=jnp.float32)
    aw = e / ssum

    rx = jnp.dot(ref_ref[:], elx_ref[:],
                 preferred_element_type=jnp.float32, precision=lax.Precision.HIGHEST)
    ry = jnp.dot(ref_ref[:], ely_ref[:],
                 preferred_element_type=jnp.float32, precision=lax.Precision.HIGHEST)

    cvec = cvec_ref[:]
    wv = cvec[0:1, :]
    hv = cvec[1:2, :]
    gx = rx * wv + ox - 0.5
    gy = ry * hv + oy - 0.5
    x0 = jnp.floor(gx)
    y0 = jnp.floor(gy)
    wx1 = gx - x0
    wx0 = 1.0 - wx1
    wy1 = gy - y0
    wy0 = 1.0 - wy1
    px = jnp.clip(x0, -1.0, wv - 1.0)
    py = jnp.clip(y0, -1.0, hv - 1.0)
    vx = (x0 == px).astype(jnp.float32)
    vy = (y0 == py).astype(jnp.float32)
    wx0 = wx0 * vx
    wx1 = wx1 * vx
    wy0 = wy0 * vy
    wy1 = wy1 * vy

    wts_ref[:] = jnp.stack(
        [aw * wy0 * wx0, aw * wy0 * wx1, aw * wy1 * wx0, aw * wy1 * wx1],
        axis=0)

    rowf = (jnp.float32(i * _QBLK)
            + lax.broadcasted_iota(jnp.int32, (_QBLK, 128), 0
                                   ).astype(jnp.float32))
    bidx = jnp.floor(rowf / jnp.float32(_Q))
    pwv = cvec[2:3, :]
    base = (bidx * 8.0 * cvec[3:4, :]
            + ((py + 1.0) * pwv + (px + 1.0)) * 8.0 + cvec[4:5, :])
    levels = []
    for lvl in range(_L):
        sl = slice(lvl * 32, (lvl + 1) * 32)
        pw8 = pwv[:, sl] * 8.0
        cat = jnp.concatenate(
            [base[:, sl], base[:, sl] + 8.0,
             base[:, sl] + pw8, base[:, sl] + pw8 + 8.0],
            axis=1)
        levels.append(cat)
    idx_ref[:] = jnp.stack(levels, axis=0).astype(jnp.int32)


def _build_index(query, reference_points, W_off, b_off, W_attn, b_attn):
    perm_off = np.zeros(2 * _L * _H * _P, np.int64)
    for hd in range(_H):
        for lvl in range(_L):
            for p in range(_P):
                for xy in range(2):
                    src = ((hd * _L + lvl) * _P + p) * 2 + xy
                    dst = xy * 128 + lvl * 32 + hd * 4 + p
                    perm_off[dst] = src
    perm_attn = np.zeros(_L * _H * _P, np.int64)
    for hd in range(_H):
        for lvl in range(_L):
            for p in range(_P):
                src = (hd * _L + lvl) * _P + p
                dst = lvl * 32 + hd * 4 + p
                perm_attn[dst] = src
    woff_t = W_off[perm_off].T
    boff = b_off[perm_off].reshape(1, 256)
    wattn_t = W_attn[perm_attn].T
    battn = b_attn[perm_attn].reshape(1, 128)
    qf = query.reshape(_NQ, _D)
    rf = reference_points.reshape(_NQ, 2 * _L)
    cvec = np.zeros((8, 128), np.float32)
    cvec[0], cvec[1], cvec[2] = _WVEC, _HVEC, _PWVEC
    cvec[3], cvec[4] = _RVEC, _HEADVEC
    return pl.pallas_call(
        _index_kernel,
        grid=(_NBLK,),
        in_specs=[
            pl.BlockSpec((_QBLK, _D), lambda i: (i, 0)),
            pl.BlockSpec((_QBLK, 2 * _L), lambda i: (i, 0)),
            pl.BlockSpec((_D, _D), lambda i: (0, 0)),
            pl.BlockSpec((1, _D), lambda i: (0, 0)),
            pl.BlockSpec((_D, 128), lambda i: (0, 0)),
            pl.BlockSpec((1, 128), lambda i: (0, 0)),
            pl.BlockSpec((128, 128), lambda i: (0, 0)),
            pl.BlockSpec((2 * _L, 128), lambda i: (0, 0)),
            pl.BlockSpec((2 * _L, 128), lambda i: (0, 0)),
            pl.BlockSpec((8, 128), lambda i: (0, 0)),
        ],
        out_specs=[
            pl.BlockSpec((_L, _QBLK, 128), lambda i: (0, i, 0)),
            pl.BlockSpec((4, _QBLK, 128), lambda i: (0, i, 0)),
        ],
        out_shape=[
            jax.ShapeDtypeStruct((_L, _NQ, 128), jnp.int32),
            jax.ShapeDtypeStruct((4, _NQ, 128), jnp.float32),
        ],
    )(qf, rf, woff_t, boff, wattn_t, battn,
      _G, _ELX, _ELY, cvec)


_GRP = 9
_NGRP = _ROWS_PER_W // _GRP


def _sc_kernel(t0, t1, t2, t3, idx_hbm, wts_hbm, out_hbm,
               pkt_idx, pkt_wts, dst_v, out_v, sem_pkt, sem_g, sem_out):
    tabs = (t0, t1, t2, t3)
    wid = lax.axis_index("s") * 2 + lax.axis_index("c")
    qbase = wid * _ROWS_PER_W

    def issue_pkt(g, slot):
        for l in range(_L):
            pltpu.async_copy(idx_hbm.at[l, pl.ds(qbase + g * _GRP, _GRP)],
                             pkt_idx.at[slot, l], sem_pkt)
        for c in range(4):
            pltpu.async_copy(wts_hbm.at[c, pl.ds(qbase + g * _GRP, _GRP)],
                             pkt_wts.at[slot, c], sem_pkt)

    def wait_pkt(slot):
        for l in range(_L):
            pltpu.make_async_copy(idx_hbm.at[l, pl.ds(qbase, _GRP)],
                                  pkt_idx.at[slot, l], sem_pkt).wait()
        for c in range(4):
            pltpu.make_async_copy(wts_hbm.at[c, pl.ds(qbase, _GRP)],
                                  pkt_wts.at[slot, c], sem_pkt).wait()

    def issue_gathers(pslot, prow, gslot):
        for l in range(_L):
            pltpu.async_copy(tabs[l].at[pkt_idx.at[pslot, l, prow]],
                             dst_v.at[gslot, l], sem_g)

    def wait_gathers(gslot):
        for l in range(_L):
            pltpu.make_async_copy(tabs[l].at[pkt_idx.at[0, 0, l]],
                                  dst_v.at[gslot, l], sem_g).wait()

    def issue_out(g, oslot):
        pltpu.async_copy(out_v.at[oslot],
                         out_hbm.at[pl.ds((qbase + g * _GRP) * 2,
                                          _GRP * 2)], sem_out)

    def wait_out(oslot):
        pltpu.make_async_copy(out_v.at[oslot],
                              out_hbm.at[pl.ds(qbase * 2, _GRP * 2)],
                              sem_out).wait()

    issue_pkt(0, 0)
    wait_pkt(0)
    issue_pkt(1, 1)
    issue_gathers(0, 0, 0)

    def chunk(i, _):
        g = lax.div(i, _GRP)
        r9 = lax.rem(i, _GRP)
        gs = lax.rem(i, 2)
        ps = lax.rem(g, 3)
        oslot = lax.rem(g, 2)

        @pl.when(r9 == 0)
        def _():
            @pl.when(g + 2 < _NGRP)
            def _():
                issue_pkt(g + 2, lax.rem(g + 2, 3))

            @pl.when(g + 1 < _NGRP)
            def _():
                wait_pkt(lax.rem(g + 1, 3))

            @pl.when(g >= 2)
            def _():
                wait_out(oslot)

        n = i + 1

        @pl.when(n < _ROWS_PER_W)
        def _():
            issue_gathers(lax.rem(lax.div(n, _GRP), 3), lax.rem(n, _GRP),
                          1 - gs)

        wait_gathers(gs)

        wvecs = {}
        for lvl in range(_L):
            for c in range(4):
                for hg in range(2):
                    wvecs[(lvl, c, hg)] = pkt_wts[
                        ps, c, r9, pl.ds(lvl * 32 + hg * 16, 16)]
        for hd in range(_H):
            hg, j = hd // 4, hd % 4
            acc0 = jnp.zeros((16,), jnp.float32)
            acc1 = jnp.zeros((16,), jnp.float32)
            for lvl in range(_L):
                for p in range(_P):
                    for c in range(4):
                        r = c * 32 + hd * 4 + p
                        w = wvecs[(lvl, c, hg)][j * 4 + p]
                        wb = jnp.full((16,), w, jnp.float32)
                        acc0 = acc0 + wb * dst_v[gs, lvl, r, pl.ds(0, 16)]
                        acc1 = acc1 + wb * dst_v[gs, lvl, r, pl.ds(16, 16)]
            orow = r9 * 2 + hd // 4
            ocol = (hd % 4) * 32
            out_v[oslot, orow, pl.ds(ocol, 16)] = acc0
            out_v[oslot, orow, pl.ds(ocol + 16, 16)] = acc1

        @pl.when(r9 == _GRP - 1)
        def _():
            issue_out(g, oslot)

        return 0

    lax.fori_loop(0, _ROWS_PER_W, chunk, 0)
    wait_out(lax.rem(_NGRP - 2, 2))
    wait_out(lax.rem(_NGRP - 1, 2))


def _sc_gather(tables, idx, wts):
    mesh = plsc.VectorSubcoreMesh(core_axis_name="c", subcore_axis_name="s")
    run = pl.kernel(
        _sc_kernel,
        out_type=jax.ShapeDtypeStruct((_NQ * 2, 128), jnp.float32),
        mesh=mesh,
        scratch_types=[
            pltpu.VMEM((3, _L, _GRP, 128), jnp.int32),
            pltpu.VMEM((3, 4, _GRP, 128), jnp.float32),
            pltpu.VMEM((2, _L, 128, _HD), jnp.float32),
            pltpu.VMEM((2, _GRP * 2, 128), jnp.float32),
            pltpu.SemaphoreType.DMA,
            pltpu.SemaphoreType.DMA,
            pltpu.SemaphoreType.DMA,
        ],
        compiler_params=pltpu.CompilerParams(use_tc_tiling_on_sc=False),
    )
    return run(*tables, idx, wts)


def _proj_kernel(x_ref, w_ref, b_ref, o_ref):
    x = x_ref[:].reshape(_QBLK, _D)
    o_ref[:] = (jnp.dot(x, w_ref[:], preferred_element_type=jnp.float32, precision=lax.Precision.HIGHEST)
                + b_ref[:])


def _out_proj(sampled, W_out, b_out):
    x = sampled
    out = pl.pallas_call(
        _proj_kernel,
        grid=(_NBLK,),
        in_specs=[
            pl.BlockSpec((_QBLK * 2, 128), lambda i: (i, 0)),
            pl.BlockSpec((_D, _D), lambda i: (0, 0)),
            pl.BlockSpec((1, _D), lambda i: (0, 0)),
        ],
        out_specs=pl.BlockSpec((_QBLK, _D), lambda i: (i, 0)),
        out_shape=jax.ShapeDtypeStruct((_NQ, _D), jnp.float32),
    )(x, W_out.T, b_out.reshape(1, _D))
    return out.reshape(_B, _Q, _D)


def kernel(query, reference_points, value, spatial_shapes, level_start_index,
           W_value, b_value, W_off, b_off, W_attn, b_attn, W_out, b_out):
    del spatial_shapes, level_start_index
    tables = _build_tables(value, W_value, b_value)
    idx, wts = _build_index(query, reference_points, W_off, b_off,
                            W_attn, b_attn)
    sampled = _sc_gather(tables, idx, wts)
    return _out_proj(sampled, W_out, b_out)


# speedup vs baseline: 136.9574x; 1.1685x over previous
import functools

import numpy as np
import jax
import jax.numpy as jnp
from jax import lax
from jax.experimental import pallas as pl
from jax.experimental.pallas import tpu as pltpu
from jax.experimental.pallas import tpu_sc as plsc

_SPATIAL = [(64, 64), (32, 32), (16, 16), (8, 8)]
_B, _Q, _D = 8, 900, 256
_H, _HD = 8, 32
_L, _P = 4, 4
_S = sum(h * w for h, w in _SPATIAL)
_NQ = _B * _Q
_N = _NQ * _H
_QBLK = 400
_NBLK = _NQ // _QBLK
_RL = [(h + 2) * (w + 2) for h, w in _SPATIAL]
_NW = 32
_ROWS_PER_W = _NQ // _NW

_COL_L = np.repeat(np.arange(_L), 32)
_COL_HEAD = np.tile(np.repeat(np.arange(_H), _P), _L)
_WVEC = np.array([_SPATIAL[l][1] for l in _COL_L], np.float32)
_HVEC = np.array([_SPATIAL[l][0] for l in _COL_L], np.float32)
_PWVEC = _WVEC + 2.0
_RVEC = np.array([_RL[l] for l in _COL_L], np.float32)
_HEADVEC = _COL_HEAD.astype(np.float32)

_ELX = np.zeros((2 * _L, 128), np.float32)
_ELY = np.zeros((2 * _L, 128), np.float32)
for _c in range(128):
    _ELX[2 * _COL_L[_c] + 0, _c] = 1.0
    _ELY[2 * _COL_L[_c] + 1, _c] = 1.0

_G = np.zeros((128, 128), np.float32)
for _i in range(128):
    for _j in range(128):
        if _COL_HEAD[_i] == _COL_HEAD[_j]:
            _G[_i, _j] = 1.0

_SINT = np.zeros((128, 128), np.float32)
for _c in range(4):
    for _hp in range(32):
        _SINT[_c * 32 + _hp, _hp * 4 + _c] = 1.0


def _tables_kernel(val_ref, wvt_ref, bv_ref, *out_refs):
    v = jnp.dot(val_ref[0], wvt_ref[:], preferred_element_type=jnp.float32)
    v = v + bv_ref[:]
    start = 0
    for lvl, (hh, ww) in enumerate(_SPATIAL):
        vl = v[start:start + hh * ww].reshape(hh, ww, _D)
        start += hh * ww
        zc = jnp.zeros((hh, 1, _D), jnp.float32)
        vl = jnp.concatenate([zc, vl, zc], axis=1)
        zr = jnp.zeros((1, ww + 2, _D), jnp.float32)
        vl = jnp.concatenate([zr, vl, zr], axis=0)
        out_refs[lvl][:] = vl.reshape((hh + 2) * (ww + 2) * 2, 128)


def _build_tables(value, W_value, b_value):
    wvt = W_value.T
    bv = b_value.reshape(1, _D)
    out_shapes = [jax.ShapeDtypeStruct((_B * r * 2, 128), jnp.float32)
                  for r in _RL]
    out_specs = [pl.BlockSpec((r * 2, 128), lambda b: (b, 0))
                 for r in _RL]
    tables = pl.pallas_call(
        _tables_kernel,
        grid=(_B,),
        in_specs=[
            pl.BlockSpec((1, _S, _D), lambda b: (b, 0, 0)),
            pl.BlockSpec((_D, _D), lambda b: (0, 0)),
            pl.BlockSpec((1, _D), lambda b: (0, 0)),
        ],
        out_specs=out_specs,
        out_shape=out_shapes,
    )(value, wvt, bv)
    return [t.reshape(_B * r * _H, _HD) for t, r in zip(tables, _RL)]


def _index_kernel(q_ref, ref_ref, woff_ref, boff_ref, wattn_ref, battn_ref,
                  g_ref, elx_ref, ely_ref, cvec_ref,
                  idx_ref, wts_ref):
    i = pl.program_id(0)
    qb = q_ref[:]
    off = jnp.dot(qb, woff_ref[:], preferred_element_type=jnp.float32)
    off = off + boff_ref[:]
    ox = off[:, :128]
    oy = off[:, 128:]
    logit = jnp.dot(qb, wattn_ref[:], preferred_element_type=jnp.float32)
    logit = logit + battn_ref[:]
    e = jnp.exp(logit)
    ssum = jnp.dot(e, g_ref[:], preferred_element_type=jnp.float32)
    aw = e / ssum

    rx = jnp.dot(ref_ref[:], elx_ref[:],
                 preferred_element_type=jnp.float32, precision=lax.Precision.HIGHEST)
    ry = jnp.dot(ref_ref[:], ely_ref[:],
                 preferred_element_type=jnp.float32, precision=lax.Precision.HIGHEST)

    cvec = cvec_ref[:]
    wv = cvec[0:1, :]
    hv = cvec[1:2, :]
    gx = rx * wv + ox - 0.5
    gy = ry * hv + oy - 0.5
    x0 = jnp.floor(gx)
    y0 = jnp.floor(gy)
    wx1 = gx - x0
    wx0 = 1.0 - wx1
    wy1 = gy - y0
    wy0 = 1.0 - wy1
    px = jnp.clip(x0, -1.0, wv - 1.0)
    py = jnp.clip(y0, -1.0, hv - 1.0)
    vx = (x0 == px).astype(jnp.float32)
    vy = (y0 == py).astype(jnp.float32)
    wx0 = wx0 * vx
    wx1 = wx1 * vx
    wy0 = wy0 * vy
    wy1 = wy1 * vy

    wts_ref[:] = jnp.stack(
        [aw * wy0 * wx0, aw * wy0 * wx1, aw * wy1 * wx0, aw * wy1 * wx1],
        axis=0)

    rowf = (jnp.float32(i * _QBLK)
            + lax.broadcasted_iota(jnp.int32, (_QBLK, 128), 0
                                   ).astype(jnp.float32))
    bidx = jnp.floor(rowf / jnp.float32(_Q))
    pwv = cvec[2:3, :]
    base = (bidx * 8.0 * cvec[3:4, :]
            + ((py + 1.0) * pwv + (px + 1.0)) * 8.0 + cvec[4:5, :])
    levels = []
    for lvl in range(_L):
        sl = slice(lvl * 32, (lvl + 1) * 32)
        pw8 = pwv[:, sl] * 8.0
        cat = jnp.concatenate(
            [base[:, sl], base[:, sl] + 8.0,
             base[:, sl] + pw8, base[:, sl] + pw8 + 8.0],
            axis=1)
        levels.append(cat)
    idx_ref[:] = jnp.stack(levels, axis=0).astype(jnp.int32)


def _build_index(query, reference_points, W_off, b_off, W_attn, b_attn):
    perm_off = np.zeros(2 * _L * _H * _P, np.int64)
    for hd in range(_H):
        for lvl in range(_L):
            for p in range(_P):
                for xy in range(2):
                    src = ((hd * _L + lvl) * _P + p) * 2 + xy
                    dst = xy * 128 + lvl * 32 + hd * 4 + p
                    perm_off[dst] = src
    perm_attn = np.zeros(_L * _H * _P, np.int64)
    for hd in range(_H):
        for lvl in range(_L):
            for p in range(_P):
                src = (hd * _L + lvl) * _P + p
                dst = lvl * 32 + hd * 4 + p
                perm_attn[dst] = src
    woff_t = W_off[perm_off].T
    boff = b_off[perm_off].reshape(1, 256)
    wattn_t = W_attn[perm_attn].T
    battn = b_attn[perm_attn].reshape(1, 128)
    qf = query.reshape(_NQ, _D)
    rf = reference_points.reshape(_NQ, 2 * _L)
    cvec = np.zeros((8, 128), np.float32)
    cvec[0], cvec[1], cvec[2] = _WVEC, _HVEC, _PWVEC
    cvec[3], cvec[4] = _RVEC, _HEADVEC
    return pl.pallas_call(
        _index_kernel,
        grid=(_NBLK,),
        in_specs=[
            pl.BlockSpec((_QBLK, _D), lambda i: (i, 0)),
            pl.BlockSpec((_QBLK, 2 * _L), lambda i: (i, 0)),
            pl.BlockSpec((_D, _D), lambda i: (0, 0)),
            pl.BlockSpec((1, _D), lambda i: (0, 0)),
            pl.BlockSpec((_D, 128), lambda i: (0, 0)),
            pl.BlockSpec((1, 128), lambda i: (0, 0)),
            pl.BlockSpec((128, 128), lambda i: (0, 0)),
            pl.BlockSpec((2 * _L, 128), lambda i: (0, 0)),
            pl.BlockSpec((2 * _L, 128), lambda i: (0, 0)),
            pl.BlockSpec((8, 128), lambda i: (0, 0)),
        ],
        out_specs=[
            pl.BlockSpec((_L, _QBLK, 128), lambda i: (0, i, 0)),
            pl.BlockSpec((4, _QBLK, 128), lambda i: (0, i, 0)),
        ],
        out_shape=[
            jax.ShapeDtypeStruct((_L, _NQ, 128), jnp.int32),
            jax.ShapeDtypeStruct((4, _NQ, 128), jnp.float32),
        ],
    )(qf, rf, woff_t, boff, wattn_t, battn,
      _G, _ELX, _ELY, cvec)


_GRP = 9
_NGRP = _ROWS_PER_W // _GRP


def _sc_kernel(t0, t1, t2, t3, idx_hbm, wts_hbm, out_hbm,
               pkt_idx, pkt_wts, dst_v, out_v, sem_pkt, sem_g, sem_out):
    tabs = (t0, t1, t2, t3)
    wid = lax.axis_index("s") * 2 + lax.axis_index("c")
    qbase = wid * _ROWS_PER_W

    def issue_pkt(g, slot):
        for l in range(_L):
            pltpu.async_copy(idx_hbm.at[l, pl.ds(qbase + g * _GRP, _GRP)],
                             pkt_idx.at[slot, l], sem_pkt)
        for c in range(4):
            pltpu.async_copy(wts_hbm.at[c, pl.ds(qbase + g * _GRP, _GRP)],
                             pkt_wts.at[slot, c], sem_pkt)

    def wait_pkt(slot):
        for l in range(_L):
            pltpu.make_async_copy(idx_hbm.at[l, pl.ds(qbase, _GRP)],
                                  pkt_idx.at[slot, l], sem_pkt).wait()
        for c in range(4):
            pltpu.make_async_copy(wts_hbm.at[c, pl.ds(qbase, _GRP)],
                                  pkt_wts.at[slot, c], sem_pkt).wait()

    def issue_gathers(pslot, prow, gslot):
        for l in range(_L):
            pltpu.async_copy(tabs[l].at[pkt_idx.at[pslot, l, prow]],
                             dst_v.at[gslot, l], sem_g)

    def wait_gathers(gslot):
        for l in range(_L):
            pltpu.make_async_copy(tabs[l].at[pkt_idx.at[0, 0, l]],
                                  dst_v.at[gslot, l], sem_g).wait()

    def issue_out(g, oslot):
        pltpu.async_copy(out_v.at[oslot],
                         out_hbm.at[pl.ds((qbase + g * _GRP) * 2,
                                          _GRP * 2)], sem_out)

    def wait_out(oslot):
        pltpu.make_async_copy(out_v.at[oslot],
                              out_hbm.at[pl.ds(qbase * 2, _GRP * 2)],
                              sem_out).wait()

    issue_pkt(0, 0)
    wait_pkt(0)
    issue_pkt(1, 1)
    issue_gathers(0, 0, 0)
    issue_gathers(0, 1, 1)

    def chunk(i, _):
        g = lax.div(i, _GRP)
        r9 = lax.rem(i, _GRP)
        gs = lax.rem(i, 3)
        ps = lax.rem(g, 3)
        oslot = lax.rem(g, 2)

        @pl.when(r9 == 0)
        def _():
            @pl.when(g + 2 < _NGRP)
            def _():
                issue_pkt(g + 2, lax.rem(g + 2, 3))

            @pl.when(g + 1 < _NGRP)
            def _():
                wait_pkt(lax.rem(g + 1, 3))

            @pl.when(g >= 2)
            def _():
                wait_out(oslot)

        n = i + 2

        @pl.when(n < _ROWS_PER_W)
        def _():
            issue_gathers(lax.rem(lax.div(n, _GRP), 3), lax.rem(n, _GRP),
                          lax.rem(n, 3))

        wait_gathers(gs)

        wvecs = {}
        for lvl in range(_L):
            for c in range(4):
                for hg in range(2):
                    wvecs[(lvl, c, hg)] = pkt_wts[
                        ps, c, r9, pl.ds(lvl * 32 + hg * 16, 16)]
        for hd in range(_H):
            hg, j = hd // 4, hd % 4
            acc0 = jnp.zeros((16,), jnp.float32)
            acc1 = jnp.zeros((16,), jnp.float32)
            for lvl in range(_L):
                for p in range(_P):
                    for c in range(4):
                        r = c * 32 + hd * 4 + p
                        w = wvecs[(lvl, c, hg)][j * 4 + p]
                        wb = jnp.full((16,), w, jnp.float32)
                        acc0 = acc0 + wb * dst_v[gs, lvl, r, pl.ds(0, 16)]
                        acc1 = acc1 + wb * dst_v[gs, lvl, r, pl.ds(16, 16)]
            orow = r9 * 2 + hd // 4
            ocol = (hd % 4) * 32
            out_v[oslot, orow, pl.ds(ocol, 16)] = acc0
            out_v[oslot, orow, pl.ds(ocol + 16, 16)] = acc1

        @pl.when(r9 == _GRP - 1)
        def _():
            issue_out(g, oslot)

        return 0

    lax.fori_loop(0, _ROWS_PER_W, chunk, 0)
    wait_out(lax.rem(_NGRP - 2, 2))
    wait_out(lax.rem(_NGRP - 1, 2))


def _sc_gather(tables, idx, wts):
    mesh = plsc.VectorSubcoreMesh(core_axis_name="c", subcore_axis_name="s")
    run = pl.kernel(
        _sc_kernel,
        out_type=jax.ShapeDtypeStruct((_NQ * 2, 128), jnp.float32),
        mesh=mesh,
        scratch_types=[
            pltpu.VMEM((3, _L, _GRP, 128), jnp.int32),
            pltpu.VMEM((3, 4, _GRP, 128), jnp.float32),
            pltpu.VMEM((3, _L, 128, _HD), jnp.float32),
            pltpu.VMEM((2, _GRP * 2, 128), jnp.float32),
            pltpu.SemaphoreType.DMA,
            pltpu.SemaphoreType.DMA,
            pltpu.SemaphoreType.DMA,
        ],
        compiler_params=pltpu.CompilerParams(use_tc_tiling_on_sc=False),
    )
    return run(*tables, idx, wts)


def _proj_kernel(x_ref, w_ref, b_ref, o_ref):
    x = x_ref[:].reshape(_QBLK, _D)
    o_ref[:] = (jnp.dot(x, w_ref[:], preferred_element_type=jnp.float32, precision=lax.Precision.HIGHEST)
                + b_ref[:])


def _out_proj(sampled, W_out, b_out):
    x = sampled
    out = pl.pallas_call(
        _proj_kernel,
        grid=(_NBLK,),
        in_specs=[
            pl.BlockSpec((_QBLK * 2, 128), lambda i: (i, 0)),
            pl.BlockSpec((_D, _D), lambda i: (0, 0)),
            pl.BlockSpec((1, _D), lambda i: (0, 0)),
        ],
        out_specs=pl.BlockSpec((_QBLK, _D), lambda i: (i, 0)),
        out_shape=jax.ShapeDtypeStruct((_NQ, _D), jnp.float32),
    )(x, W_out.T, b_out.reshape(1, _D))
    return out.reshape(_B, _Q, _D)


def kernel(query, reference_points, value, spatial_shapes, level_start_index,
           W_value, b_value, W_off, b_off, W_attn, b_attn, W_out, b_out):
    del spatial_shapes, level_start_index
    tables = _build_tables(value, W_value, b_value)
    idx, wts = _build_index(query, reference_points, W_off, b_off,
                            W_attn, b_attn)
    sampled = _sc_gather(tables, idx, wts)
    return _out_proj(sampled, W_out, b_out)
